# Initial kernel scaffold; baseline (speedup 1.0000x reference)
import jax
import jax.numpy as jnp
from jax.experimental import pallas as pl


def kernel(codes, values):
    raise NotImplementedError("write your pallas kernel here")


# speedup vs baseline: 1.1366x; 1.1366x over previous
import functools

import jax
import jax.numpy as jnp
from jax import lax
from jax.experimental import pallas as pl
from jax.experimental.pallas import tpu as pltpu
from jax.experimental.pallas import tpu_sc as plsc

B, L, P = 16, 4096, 128
SEP = 0
LANES = 16
NC, NS = 2, 16
HALF = L // 2
CHUNKS = HALF // LANES
ROW_WORDS = P * L
WORK_WORDS = ROW_WORDS // 2
ZB = 8192
NZCOPY = WORK_WORDS // ZB
SCAT = 16


def _body(codes_hbm, values_hbm, padded_hbm, lengths_hbm,
          codes_v, vals_v, codes_p, dest_v, sval_v, hist_v, tmp_v, zero_v,
          shared_hist, zsem, ssem):
    c_id = lax.axis_index("c")
    s_id = lax.axis_index("s")
    w = c_id * NS + s_id
    b = w // 2
    h = s_id % 2
    b_local = s_id // 2
    row_off = b * L
    row_base = b * ROW_WORDS
    safe = row_base + ROW_WORDS - 1

    pl.when(h == 1)(lambda: pltpu.sync_copy(
        codes_hbm.at[pl.ds(row_off, HALF)], codes_p))
    pltpu.sync_copy(codes_hbm.at[pl.ds(row_off + h * HALF, HALF)], codes_v)
    pltpu.sync_copy(values_hbm.at[pl.ds(row_off + h * HALF, HALF)], vals_v)

    zeros16f = jnp.zeros((LANES,), jnp.float32)
    zeros16i = jnp.zeros((LANES,), jnp.int32)

    def _zb(k, carry):
        zero_v[pl.ds(k * LANES, LANES)] = zeros16f
        return carry
    lax.fori_loop(0, ZB // LANES, _zb, 0)
    for i in range(P // LANES):
        hist_v[pl.ds(i * LANES, LANES)] = zeros16i

    zdescs = [
        pltpu.async_copy(
            zero_v,
            padded_hbm.at[pl.ds(w * WORK_WORDS + j * ZB, ZB)],
            zsem)
        for j in range(NZCOPY)
    ]

    def _prescan(k, carry):
        cnt, stv = carry
        c = codes_p[pl.ds(k * LANES, LANES)]
        m = c == SEP
        cnt = cnt + jnp.sum(jnp.where(m, 1, 0).astype(jnp.int32))
        iv = k * LANES + lax.iota(jnp.int32, LANES)
        spos = jnp.where(m, iv + 1, 0)
        return cnt, jnp.maximum(stv, jnp.max(spos))

    cnt0, stv0 = lax.fori_loop(
        0, h * CHUNKS, _prescan,
        (jnp.zeros((LANES,), jnp.int32), jnp.int32(0)))

    ones16 = jnp.ones((LANES,), jnp.int32)
    base_i = h * HALF

    def _scan(k, carry):
        cnt, stv = carry
        c = codes_v[pl.ds(k * LANES, LANES)]
        v = vals_v[pl.ds(k * LANES, LANES)]
        m = c == SEP
        si = jnp.where(m, 1, 0).astype(jnp.int32)
        cs = plsc.cumsum(si)
        piece = cnt + (cs - si)
        iv = base_i + k * LANES + lax.iota(jnp.int32, LANES)
        spos = jnp.where(m, iv + 1, 0)
        start = jnp.maximum(plsc.cummax(spos), stv)
        pos = iv - start
        inb = jnp.logical_and(jnp.logical_not(m), piece < P)
        dest = jnp.where(inb, row_base + piece * L + pos, safe)
        val = jnp.where(inb, v, 0.0)
        dest_v[k // 8, pl.ds((k % 8) * LANES, LANES)] = dest
        sval_v[k // 8, pl.ds((k % 8) * LANES, LANES)] = val
        plsc.addupdate_scatter(hist_v, [piece], ones16, mask=inb)
        cnt = cnt + jnp.sum(si)
        return cnt, jnp.maximum(stv, jnp.max(spos))

    lax.fori_loop(0, CHUNKS, _scan, (cnt0, stv0))

    pl.when(h == 0)(lambda: pltpu.sync_copy(hist_v, shared_hist.at[b_local]))

    for d in zdescs:
        d.wait()
    plsc.subcore_barrier()

    sdescs = [
        pltpu.async_copy(sval_v.at[j], padded_hbm.at[dest_v.at[j]], ssem)
        for j in range(SCAT)
    ]
    for d in sdescs:
        d.wait()

    def _lengths():
        pltpu.sync_copy(shared_hist.at[b_local], tmp_v)
        for i in range(P // LANES):
            sl = pl.ds(i * LANES, LANES)
            hist_v[sl] = hist_v[sl] + tmp_v[sl]
        pltpu.sync_copy(hist_v, lengths_hbm.at[pl.ds(b * P, P)])
    pl.when(h == 1)(_lengths)


@jax.jit
def _split_sc(codes_f, values_f):
    mesh = plsc.VectorSubcoreMesh(
        core_axis_name="c", subcore_axis_name="s",
        num_cores=NC, num_subcores=NS)
    return pl.kernel(
        _body,
        out_type=(
            jax.ShapeDtypeStruct((B * P * L,), jnp.float32),
            jax.ShapeDtypeStruct((B * P,), jnp.int32),
        ),
        mesh=mesh,
        compiler_params=pltpu.CompilerParams(needs_layout_passes=False),
        scratch_types=[
            pltpu.VMEM((HALF,), jnp.int32),
            pltpu.VMEM((HALF,), jnp.float32),
            pltpu.VMEM((HALF,), jnp.int32),
            pltpu.VMEM((SCAT, 128), jnp.int32),
            pltpu.VMEM((SCAT, 128), jnp.float32),
            pltpu.VMEM((P,), jnp.int32),
            pltpu.VMEM((P,), jnp.int32),
            pltpu.VMEM((ZB,), jnp.float32),
            pltpu.VMEM_SHARED((NS // 2, P), jnp.int32),
            pltpu.SemaphoreType.DMA,
            pltpu.SemaphoreType.DMA,
        ],
    )(codes_f, values_f)


def kernel(codes, values):
    padded_f, lengths_f = _split_sc(codes.reshape(-1), values.reshape(-1))
    return padded_f.reshape(B, P, L), lengths_f.reshape(B, P)
